# baseline (speedup 1.0000x reference)
import functools

import jax
import jax.numpy as jnp
from jax import lax
from jax.experimental import pallas as pl
from jax.experimental.pallas import tpu as pltpu
from jax.experimental.pallas import tpu_sc as plsc

_N = 10000
_E = 320000
_D = 128
_CW = 16
_NW = 32
_EPW = _E // _NW
_K = 80
_NH = 64
_CH = 2 * _NH
_EPT = _CH * _K
_NP = 10240
_RPT = _NP // 16
_DUMMY = 10200


def _seg_body(x_hbm, src_hbm, dst_hbm, zf_hbm, zc_hbm, ones_hbm,
              pf_hbm, pc_hbm,
              acc, cacc, ones_v,
              rb0, rb1, rb2, rb3, db0, db1, db2, db3, sb0, sb1, sb2, sb3,
              g0, g1, g2, g3, f0, f1, f2, f3,
              si0, si1, si2, si3, sd0, sd1, sd2, sd3):
    cid = lax.axis_index("c")
    sid = lax.axis_index("s")
    wid = sid * 2 + cid
    rbuf = (rb0, rb1, rb2, rb3)
    dbuf = (db0, db1, db2, db3)
    sbuf = (sb0, sb1, sb2, sb3)
    gsem = (g0, g1, g2, g3)
    fsem = (f0, f1, f2, f3)
    sisem = (si0, si1, si2, si3)
    sdsem = (sd0, sd1, sd2, sd3)

    pltpu.sync_copy(ones_hbm, ones_v)
    base = sid * _RPT
    pltpu.sync_copy(zf_hbm, acc.at[pl.ds(base, _RPT)])
    pltpu.sync_copy(zc_hbm, cacc.at[pl.ds(base, _RPT)])
    plsc.subcore_barrier()

    def issue_idx(h, s):
        pltpu.async_copy(src_hbm.at[wid].at[h], sbuf[s], sisem[s])
        pltpu.async_copy(dst_hbm.at[wid].at[h], dbuf[s], sdsem[s])

    def wait_idx(s):
        pltpu.make_async_copy(src_hbm.at[wid].at[0], sbuf[s], sisem[s]).wait()
        pltpu.make_async_copy(dst_hbm.at[wid].at[0], dbuf[s], sdsem[s]).wait()

    def issue_gather(rb, s, b):
        pltpu.async_copy(x_hbm.at[sbuf[s].at[pl.ds(b * _K, _K)]],
                         rbuf[rb], gsem[rb])

    def wait_gather(rb, s, b):
        pltpu.make_async_copy(x_hbm.at[sbuf[s].at[pl.ds(b * _K, _K)]],
                              rbuf[rb], gsem[rb]).wait()

    def issue_fscatter(rb, s, b):
        pltpu.async_copy(rbuf[rb], acc.at[dbuf[s].at[b]], fsem[rb],
                         add=True)

    def wait_fscatter(rb, s, b):
        pltpu.make_async_copy(rbuf[rb], acc.at[dbuf[s].at[b]],
                              fsem[rb]).wait()

    issue_idx(0, 0)

    def group(g, carry):
        h0 = g * 4
        for hh in range(4):
            pair = hh % 2
            s = hh
            for b in range(2):
                if hh >= 2:
                    wait_fscatter(2 * pair + b, (hh - 2) % 4, b)
                else:
                    @pl.when(g > 0)
                    def _():
                        wait_fscatter(2 * pair + b, (hh - 2) % 4, b)
            wait_idx(s)
            for b in range(2):
                issue_gather(2 * pair + b, s, b)
            @pl.when(h0 + hh + 1 < _NH)
            def _():
                issue_idx(h0 + hh + 1, (hh + 1) % 4)
            for b in range(2):
                wait_gather(2 * pair + b, s, b)
                issue_fscatter(2 * pair + b, s, b)
                pltpu.sync_copy(ones_v, cacc.at[dbuf[s].at[b]], add=True)
        return carry

    lax.fori_loop(0, _NH // 4, group, 0)

    for hh in (2, 3):
        for b in range(2):
            wait_fscatter(2 * (hh % 2) + b, hh, b)
    plsc.subcore_barrier()

    pltpu.sync_copy(acc.at[pl.ds(base, _RPT)],
                    pf_hbm.at[cid].at[pl.ds(base, _RPT)])
    pltpu.sync_copy(cacc.at[pl.ds(base, _RPT)],
                    pc_hbm.at[cid].at[pl.ds(base, _RPT)])


@functools.cache
def _make_seg():
  return pl.kernel(
    _seg_body,
    out_type=(jax.ShapeDtypeStruct((2, _NP, _D), jnp.float32),
              jax.ShapeDtypeStruct((2, _NP), jnp.float32)),
    mesh=plsc.VectorSubcoreMesh(core_axis_name="c", subcore_axis_name="s"),
    scratch_types=(
        [pltpu.VMEM_SHARED((_NP, _D), jnp.float32),
         pltpu.VMEM_SHARED((_NP,), jnp.float32),
         pltpu.VMEM((_K,), jnp.float32)]
        + [pltpu.VMEM((_K, _D), jnp.float32)] * 4
        + [pltpu.VMEM((2, _K), jnp.int32)] * 4
        + [pltpu.VMEM((2 * _K,), jnp.int32)] * 4
        + [pltpu.SemaphoreType.DMA] * 16
    ),
  )


def _post_body(pf_ref, cnt_ref, x_ref, wl_ref, bl_ref, wr_ref, ws_ref,
               bs_ref, al_ref, rs_ref, out_ref):
    seg = pf_ref[0] + pf_ref[1]
    mean = seg / jnp.maximum(cnt_ref[...], 1.0)
    x = x_ref[...]
    pre = (jnp.dot(mean, wl_ref[...], preferred_element_type=jnp.float32)
           + bl_ref[...]
           + jnp.dot(x, wr_ref[...], preferred_element_type=jnp.float32))
    h = jnp.maximum(pre, 0.0) + x
    sc = jnp.dot(h, ws_ref[...], preferred_element_type=jnp.float32) + bs_ref[...]
    a = jax.nn.sigmoid(al_ref[...])
    out_ref[...] = a * rs_ref[...] + (1.0 - a) * sc


_post = pl.pallas_call(
    _post_body,
    out_shape=jax.ShapeDtypeStruct((_N, 1), jnp.float32),
    grid=(1,),
    in_specs=[
        pl.BlockSpec((2, _N, _D), lambda i: (0, 0, 0)),
        pl.BlockSpec((_N, 1), lambda i: (0, 0)),
        pl.BlockSpec((_N, _D), lambda i: (0, 0)),
        pl.BlockSpec((_D, _D), lambda i: (0, 0)),
        pl.BlockSpec((1, _D), lambda i: (0, 0)),
        pl.BlockSpec((_D, _D), lambda i: (0, 0)),
        pl.BlockSpec((_D, 1), lambda i: (0, 0)),
        pl.BlockSpec((1, 1), lambda i: (0, 0)),
        pl.BlockSpec((1, 1), lambda i: (0, 0)),
        pl.BlockSpec((_N, 1), lambda i: (0, 0)),
    ],
    out_specs=pl.BlockSpec((_N, 1), lambda i: (0, 0)),
)


@jax.jit
def kernel(x, edge_index, reranker_scores, W_l, b_l, W_r, w_score, b_score,
           alpha):
    e2 = edge_index.reshape(2, _NW, _EPW)
    pad = _EPT - _EPW
    src = jnp.concatenate(
        [e2[0], jnp.zeros((_NW, pad), jnp.int32)], axis=1
    ).reshape(_NW, _NH, 2 * _K)
    dst = jnp.concatenate(
        [e2[1], jnp.full((_NW, pad), _DUMMY, jnp.int32)], axis=1
    ).reshape(_NW, _NH, 2, _K)
    zf = jnp.zeros((_RPT, _D), jnp.float32)
    zc = jnp.zeros((_RPT,), jnp.float32)
    ones1 = jnp.ones((_K,), jnp.float32)
    pf, pcnt = _make_seg()(x, src, dst, zf, zc, ones1)
    cnt = (pcnt[0, :_N] + pcnt[1, :_N]).reshape(_N, 1)
    out = _post(pf, cnt, x, W_l, b_l.reshape(1, _D), W_r, w_score,
                b_score.reshape(1, 1), alpha.reshape(1, 1),
                reranker_scores.reshape(_N, 1))
    return out[:, 0]


# speedup vs baseline: 2.4272x; 2.4272x over previous
import functools

import jax
import jax.numpy as jnp
from jax import lax
from jax.experimental import pallas as pl
from jax.experimental.pallas import tpu as pltpu
from jax.experimental.pallas import tpu_sc as plsc

_N = 10000
_E = 320000
_D = 128
_CW = 16
_NW = 32
_EPW = _E // _NW
_K = 125
_NCH = _EPW // _K
_NBUF = 4
_NP = 10240
_RPT = _NP // 16


def _seg_body(x_hbm, src_hbm, dst_hbm, zf_hbm, zc_hbm, ones_hbm,
              pf_hbm, pc_hbm,
              acc, cacc, dst_v, ones_v, i0, i1, i2, i3, b0, b1,
              si0, si1, si2, si3, s0, s1, ss0, ss1, cs0, cs1):
    cid = lax.axis_index("c")
    sid = lax.axis_index("s")
    wid = sid * 2 + cid
    ipair = ((i0, i1), (i2, i3))
    bufs = (b0, b1)
    ispair = ((si0, si1), (si2, si3))
    sems = (s0, s1)
    ssems = (ss0, ss1)
    csems = (cs0, cs1)

    pltpu.sync_copy(dst_hbm.at[wid], dst_v)
    pltpu.sync_copy(ones_hbm, ones_v)

    base = sid * _RPT
    pltpu.sync_copy(zf_hbm, acc.at[pl.ds(base, _RPT)])
    pltpu.sync_copy(zc_hbm, cacc.at[pl.ds(base, _RPT)])
    plsc.subcore_barrier()

    for b in range(2):
        pltpu.async_copy(src_hbm.at[wid].at[b], ipair[0][b], ispair[0][b])

    def group(g, carry):
        for half in range(2):
            c0 = g * 4 + half * 2
            hr = []
            for b in range(2):
                pltpu.make_async_copy(src_hbm.at[wid].at[0], ipair[half][b],
                                      ispair[half][b]).wait()
                hr.append(pltpu.async_copy(x_hbm.at[ipair[half][b]], bufs[b],
                                           sems[b]))
            nxt = 1 - half
            @pl.when(c0 + 2 < _NCH)
            def _():
                for b in range(2):
                    pltpu.async_copy(src_hbm.at[wid].at[c0 + 2 + b],
                                     ipair[nxt][b], ispair[nxt][b])
            hs = []
            for b in range(2):
                hr[b].wait()
                hs.append(pltpu.async_copy(bufs[b], acc.at[dst_v.at[c0 + b]],
                                           ssems[b], add=True))
                hs.append(pltpu.async_copy(ones_v, cacc.at[dst_v.at[c0 + b]],
                                           csems[b], add=True))
            for h in hs:
                h.wait()
        return carry

    lax.fori_loop(0, _NCH // 4, group, 0)
    plsc.subcore_barrier()

    pltpu.sync_copy(acc.at[pl.ds(base, _RPT)],
                    pf_hbm.at[cid].at[pl.ds(base, _RPT)])
    pltpu.sync_copy(cacc.at[pl.ds(base, _RPT)],
                    pc_hbm.at[cid].at[pl.ds(base, _RPT)])


@functools.cache
def _make_seg():
  return pl.kernel(
    _seg_body,
    out_type=(jax.ShapeDtypeStruct((2, _NP, _D), jnp.float32),
              jax.ShapeDtypeStruct((2, _NP), jnp.float32)),
    mesh=plsc.VectorSubcoreMesh(core_axis_name="c", subcore_axis_name="s"),
    scratch_types=[
        pltpu.VMEM_SHARED((_NP, _D), jnp.float32),
        pltpu.VMEM_SHARED((_NP,), jnp.float32),
        pltpu.VMEM((_NCH, _K), jnp.int32),
        pltpu.VMEM((_K,), jnp.float32),
        pltpu.VMEM((_K,), jnp.int32),
        pltpu.VMEM((_K,), jnp.int32),
        pltpu.VMEM((_K,), jnp.int32),
        pltpu.VMEM((_K,), jnp.int32),
        pltpu.VMEM((_K, _D), jnp.float32),
        pltpu.VMEM((_K, _D), jnp.float32),
        pltpu.SemaphoreType.DMA,
        pltpu.SemaphoreType.DMA,
        pltpu.SemaphoreType.DMA,
        pltpu.SemaphoreType.DMA,
        pltpu.SemaphoreType.DMA,
        pltpu.SemaphoreType.DMA,
        pltpu.SemaphoreType.DMA,
        pltpu.SemaphoreType.DMA,
        pltpu.SemaphoreType.DMA,
        pltpu.SemaphoreType.DMA,
    ],
  )


def _post_body(pf_ref, cnt_ref, x_ref, wl_ref, bl_ref, wr_ref, ws_ref,
               bs_ref, al_ref, out_ref):
    seg = pf_ref[0] + pf_ref[1]
    mean = seg / jnp.maximum(cnt_ref[...], 1.0)
    x = x_ref[...]
    pre = (jnp.dot(mean, wl_ref[...], preferred_element_type=jnp.float32)
           + bl_ref[...]
           + jnp.dot(x, wr_ref[...], preferred_element_type=jnp.float32))
    h = jnp.maximum(pre, 0.0) + x
    sc = jnp.dot(h, ws_ref[...], preferred_element_type=jnp.float32) + bs_ref[...]
    a = jax.nn.sigmoid(al_ref[...])
    out_ref[...] = (1.0 - a) * sc


_post = pl.pallas_call(
    _post_body,
    out_shape=jax.ShapeDtypeStruct((_N, 1), jnp.float32),
    grid=(1,),
    in_specs=[
        pl.BlockSpec((2, _N, _D), lambda i: (0, 0, 0)),
        pl.BlockSpec((_N, 1), lambda i: (0, 0)),
        pl.BlockSpec((_N, _D), lambda i: (0, 0)),
        pl.BlockSpec((_D, _D), lambda i: (0, 0)),
        pl.BlockSpec((1, _D), lambda i: (0, 0)),
        pl.BlockSpec((_D, _D), lambda i: (0, 0)),
        pl.BlockSpec((_D, 1), lambda i: (0, 0)),
        pl.BlockSpec((1, 1), lambda i: (0, 0)),
        pl.BlockSpec((1, 1), lambda i: (0, 0)),
    ],
    out_specs=pl.BlockSpec((_N, 1), lambda i: (0, 0)),
)


@jax.jit
def kernel(x, edge_index, reranker_scores, W_l, b_l, W_r, w_score, b_score,
           alpha):
    src = edge_index[0].reshape(_NW, _NCH, _K)
    dst = edge_index[1].reshape(_NW, _NCH, _K)
    zf = jnp.zeros((_RPT, _D), jnp.float32)
    zc = jnp.zeros((_RPT,), jnp.float32)
    ones1 = jnp.ones((_K,), jnp.float32)
    pf, pcnt = _make_seg()(x, src, dst, zf, zc, ones1)
    cnt = (pcnt[0, :_N] + pcnt[1, :_N]).reshape(_N, 1)
    gnn = _post(pf, cnt, x, W_l, b_l.reshape(1, _D), W_r, w_score,
                b_score.reshape(1, 1), alpha.reshape(1, 1))
    return jax.nn.sigmoid(alpha) * reranker_scores + gnn[:, 0]
